# Initial kernel scaffold; baseline (speedup 1.0000x reference)
import jax
import jax.numpy as jnp
from jax.experimental import pallas as pl


def kernel(x, h, edge_attr, edge_attr_partial, edge_index, partial_goal_mask, enc_goal_params, enc_partial_params, dec_params, inf_params, prior_params):
    raise NotImplementedError("write your pallas kernel here")


# speedup vs baseline: 1.0237x; 1.0237x over previous
import jax
import jax.numpy as jnp
import numpy as np
from jax.experimental import pallas as pl

N = 50000
LAT = 32


def _apply(p, x):
    return x @ p["W"] + p["b"]


def _gnn_fwd(p, x, h, edge_attr, src, dst, n_nodes):
    hh = jax.nn.silu(_apply(p["embed"], h))
    for lp in p["layers"]:
        rel = x[src] - x[dst]
        d2 = jnp.sum(rel * rel, axis=-1, keepdims=True)
        m_in = jnp.concatenate([hh[dst], hh[src], d2, edge_attr], axis=-1)
        m = jax.nn.silu(_apply(lp["e1"], m_in))
        m = jax.nn.silu(_apply(lp["e2"], m))
        agg = jax.ops.segment_sum(m, dst, num_segments=n_nodes)
        upd = jax.nn.silu(_apply(lp["h1"], jnp.concatenate([hh, agg], axis=-1)))
        hh = hh + _apply(lp["h2"], upd)
    return _apply(p["out"], hh)


def _vae_fwd(p, z):
    hdn = jax.nn.silu(_apply(p["l1"], z))
    out = _apply(p["l2"], hdn)
    loc, raw = jnp.split(out, 2, axis=-1)
    return loc, jax.nn.softplus(raw) + 1e-4


def _reparam_kernel(q_loc_ref, q_scale_ref, eps_ref, z_ref):
    z_ref[...] = q_loc_ref[...] + q_scale_ref[...] * eps_ref[...]


def _reparam(q_loc, q_scale, eps):
    blk = pl.BlockSpec((2000, LAT), lambda i: (i, 0))
    return pl.pallas_call(
        _reparam_kernel,
        grid=(N // 2000,),
        in_specs=[blk, blk, blk],
        out_specs=blk,
        out_shape=jax.ShapeDtypeStruct(q_loc.shape, q_loc.dtype),
    )(q_loc, q_scale, eps)


def kernel(x, h, edge_attr, edge_attr_partial, edge_index, partial_goal_mask,
           enc_goal_params, enc_partial_params, dec_params, inf_params,
           prior_params):
    src = edge_index[0]
    dst = edge_index[1]
    z_goal = _gnn_fwd(enc_goal_params, x, h, edge_attr, src, dst, N)
    z_goal_partial = _gnn_fwd(enc_partial_params, partial_goal_mask[:, None] * x,
                              h, edge_attr_partial, src, dst, N)
    p_loc, p_scale = _vae_fwd(prior_params, z_goal_partial)
    q_loc, q_scale = _vae_fwd(inf_params,
                              jnp.concatenate([z_goal, z_goal_partial], axis=-1))
    eps = jax.random.normal(jax.random.key(42), q_loc.shape, jnp.float32)
    z = _reparam(q_loc, q_scale, eps)
    mu_x_sample = _gnn_fwd(dec_params,
                           jnp.concatenate([z, z_goal_partial], axis=-1),
                           h, 0.0 * edge_attr, src, dst, N)
    return (mu_x_sample, q_loc, q_scale, p_loc, p_scale)


# speedup vs baseline: 2.0739x; 2.0259x over previous
import functools

import jax
import jax.numpy as jnp
from jax import lax
from jax.experimental import pallas as pl
from jax.experimental.pallas import tpu as pltpu
from jax.experimental.pallas import tpu_sc as plsc

N = 50000
E = 800000
H_DIM = 16
LAT = 32
HID = 64
MLP_HID = 128

EPAD = 819200
NIDXROWS = EPAD // 128
NC = 2
NS = 16
NW = NC * NS
ROWS_PER_W = NIDXROWS // NW
GROUP = 8
CHUNK = GROUP * 128
NGROUPS = ROWS_PER_W // GROUP
NACC = 50048

BN = 2000
BE = 3200

_mesh = plsc.VectorSubcoreMesh(core_axis_name="c", subcore_axis_name="s")


@functools.lru_cache(maxsize=None)
def _gather2_kernel(D):
    @functools.partial(
        pl.kernel,
        mesh=_mesh,
        compiler_params=pltpu.CompilerParams(use_tc_tiling_on_sc=False),
        out_type=[jax.ShapeDtypeStruct((EPAD, D), jnp.float32),
                  jax.ShapeDtypeStruct((EPAD, D), jnp.float32)],
        scratch_types=[pltpu.VMEM((GROUP, 128), jnp.int32),
                       pltpu.VMEM((CHUNK, D), jnp.float32),
                       pltpu.SemaphoreType.DMA],
    )
    def k(table, src2d, dst2d, o_src, o_dst, idx_v, rows_v, sem):
        wid = lax.axis_index("s") * NC + lax.axis_index("c")
        row0 = wid * ROWS_PER_W
        e0 = row0 * 128

        def body(g, carry):
            for ind, out in ((src2d, o_src), (dst2d, o_dst)):
                pltpu.sync_copy(ind.at[pl.ds(row0 + g * GROUP, GROUP)], idx_v)
                copies = [
                    pltpu.async_copy(table.at[idx_v.at[j]],
                                     rows_v.at[pl.ds(j * 128, 128)], sem)
                    for j in range(GROUP)
                ]
                for cp in copies:
                    cp.wait()
                pltpu.sync_copy(rows_v, out.at[pl.ds(e0 + g * CHUNK, CHUNK)])
            return carry

        lax.fori_loop(0, NGROUPS, body, 0)

    return k


SC_GROUP = 4
SC_CHUNK = SC_GROUP * 128
SC_NGROUPS = ROWS_PER_W // SC_GROUP


@functools.lru_cache(maxsize=None)
def _scatter_kernel():
    zrows = NACC // NS

    @functools.partial(
        pl.kernel,
        mesh=_mesh,
        compiler_params=pltpu.CompilerParams(use_tc_tiling_on_sc=False),
        out_type=jax.ShapeDtypeStruct((NC, NACC, LAT), jnp.float32),
        scratch_types=[pltpu.VMEM((SC_GROUP, 128), jnp.int32),
                       pltpu.VMEM((SC_CHUNK, LAT), jnp.float32),
                       pltpu.VMEM_SHARED((NACC, LAT), jnp.float32),
                       pltpu.SemaphoreType.DMA],
    )
    def k(m, dst2d, zeros_hbm, partial, idx_v, rows_v, acc, sem):
        c = lax.axis_index("c")
        s = lax.axis_index("s")
        wid = s * NC + c
        row0 = wid * ROWS_PER_W
        e0 = row0 * 128

        pltpu.sync_copy(zeros_hbm.at[pl.ds(s * zrows, zrows)],
                        acc.at[pl.ds(s * zrows, zrows)])
        plsc.subcore_barrier()

        def body(g, carry):
            pltpu.sync_copy(dst2d.at[pl.ds(row0 + g * SC_GROUP, SC_GROUP)],
                            idx_v)
            pltpu.sync_copy(m.at[pl.ds(e0 + g * SC_CHUNK, SC_CHUNK)], rows_v)
            for j in range(SC_GROUP):
                pltpu.sync_copy(rows_v.at[pl.ds(j * 128, 128)],
                                acc.at[idx_v.at[j]], add=True)
            return carry

        lax.fori_loop(0, SC_NGROUPS, body, 0)
        plsc.subcore_barrier()
        pltpu.sync_copy(acc.at[pl.ds(s * zrows, zrows)],
                        partial.at[c, pl.ds(s * zrows, zrows)])

    return k


def _full(shape):
    return pl.BlockSpec(shape, lambda i: tuple(0 for _ in shape))


def _silu(x):
    return x * jax.nn.sigmoid(x)


def _dot(a, b):
    return jnp.dot(a, b, preferred_element_type=jnp.float32)


def _mlp1(h, W, b):
    din, dout = W.shape

    def body(h_ref, w_ref, b_ref, o_ref):
        o_ref[...] = _silu(_dot(h_ref[...], w_ref[...]) + b_ref[...])

    return pl.pallas_call(
        body,
        grid=(N // BN,),
        in_specs=[pl.BlockSpec((BN, din), lambda i: (i, 0)),
                  _full((din, dout)), _full((1, dout))],
        out_specs=pl.BlockSpec((BN, dout), lambda i: (i, 0)),
        out_shape=jax.ShapeDtypeStruct((N, dout), jnp.float32),
    )(h, W, b.reshape(1, dout))


def _proj(h, W, b):
    din, dout = W.shape

    def body(h_ref, w_ref, b_ref, o_ref):
        o_ref[...] = _dot(h_ref[...], w_ref[...]) + b_ref[...]

    return pl.pallas_call(
        body,
        grid=(N // BN,),
        in_specs=[pl.BlockSpec((BN, din), lambda i: (i, 0)),
                  _full((din, dout)), _full((1, dout))],
        out_specs=pl.BlockSpec((BN, dout), lambda i: (i, 0)),
        out_shape=jax.ShapeDtypeStruct((N, dout), jnp.float32),
    )(h, W, b.reshape(1, dout))


def _d2(xs, xd, P):
    def body(xs_ref, xd_ref, o_ref):
        r = xs_ref[...] - xd_ref[...]
        o_ref[...] = jnp.sum(r * r, axis=1, keepdims=True)

    return pl.pallas_call(
        body,
        grid=(EPAD // BE,),
        in_specs=[pl.BlockSpec((BE, P), lambda i: (i, 0)),
                  pl.BlockSpec((BE, P), lambda i: (i, 0))],
        out_specs=pl.BlockSpec((BE, 1), lambda i: (i, 0)),
        out_shape=jax.ShapeDtypeStruct((EPAD, 1), jnp.float32),
    )(xs, xd)


def _edge_mlp(hd, hs, d2, ea8, lp):
    W1 = jnp.pad(lp["e1"]["W"], ((0, 3), (0, 0)))
    b1 = lp["e1"]["b"].reshape(1, HID)
    W2 = lp["e2"]["W"]
    b2 = lp["e2"]["b"].reshape(1, LAT)
    with_ea = ea8 is not None

    def body(hd_ref, hs_ref, d2_ref, ea_ref, w1_r, b1_r, w2_r, b2_r, o_ref):
        if with_ea:
            ea = ea_ref[...]
        else:
            ea = jnp.zeros((BE, 8), jnp.float32)
        m_in = jnp.concatenate(
            [hd_ref[...], hs_ref[...], d2_ref[...], ea[:, 0:4],
             jnp.zeros((BE, 3), jnp.float32)], axis=1)
        u = _silu(_dot(m_in, w1_r[...]) + b1_r[...])
        o_ref[...] = _silu(_dot(u, w2_r[...]) + b2_r[...])

    args = [hd, hs, d2]
    in_specs = [pl.BlockSpec((BE, LAT), lambda i: (i, 0)),
                pl.BlockSpec((BE, LAT), lambda i: (i, 0)),
                pl.BlockSpec((BE, 1), lambda i: (i, 0))]
    if with_ea:
        args.append(ea8)
        in_specs.append(pl.BlockSpec((BE, 8), lambda i: (i, 0)))
    else:
        args.append(d2)
        in_specs.append(pl.BlockSpec((BE, 1), lambda i: (i, 0)))
    args += [W1, b1, W2, b2]
    in_specs += [_full((2 * LAT + 8, HID)), _full((1, HID)),
                 _full((HID, LAT)), _full((1, LAT))]

    return pl.pallas_call(
        body,
        grid=(EPAD // BE,),
        in_specs=in_specs,
        out_specs=pl.BlockSpec((BE, LAT), lambda i: (i, 0)),
        out_shape=jax.ShapeDtypeStruct((EPAD, LAT), jnp.float32),
    )(*args)


def _node_update(hh, part, lp):
    Wh1 = lp["h1"]["W"]
    bh1 = lp["h1"]["b"].reshape(1, HID)
    Wh2 = lp["h2"]["W"]
    bh2 = lp["h2"]["b"].reshape(1, LAT)
    p0 = part[0]
    p1 = part[1]

    def body(hh_ref, p0_ref, p1_ref, w1_r, b1_r, w2_r, b2_r, o_ref):
        agg = p0_ref[...] + p1_ref[...]
        cat = jnp.concatenate([hh_ref[...], agg], axis=1)
        u = _silu(_dot(cat, w1_r[...]) + b1_r[...])
        o_ref[...] = hh_ref[...] + _dot(u, w2_r[...]) + b2_r[...]

    return pl.pallas_call(
        body,
        grid=(N // BN,),
        in_specs=[pl.BlockSpec((BN, LAT), lambda i: (i, 0)),
                  pl.BlockSpec((BN, LAT), lambda i: (i, 0)),
                  pl.BlockSpec((BN, LAT), lambda i: (i, 0)),
                  _full((2 * LAT, HID)), _full((1, HID)),
                  _full((HID, LAT)), _full((1, LAT))],
        out_specs=pl.BlockSpec((BN, LAT), lambda i: (i, 0)),
        out_shape=jax.ShapeDtypeStruct((N, LAT), jnp.float32),
    )(hh, p0, p1, Wh1, bh1, Wh2, bh2)


def _softplus(x):
    return jnp.maximum(x, 0.0) + jnp.log1p(jnp.exp(-jnp.abs(x)))


def _vae_prior(zgp, p):
    L1, b1 = p["l1"]["W"], p["l1"]["b"].reshape(1, MLP_HID)
    L2, b2 = p["l2"]["W"], p["l2"]["b"].reshape(1, 2 * LAT)

    def body(z_ref, l1_r, b1_r, l2_r, b2_r, loc_ref, scale_ref):
        hdn = _silu(_dot(z_ref[...], l1_r[...]) + b1_r[...])
        o = _dot(hdn, l2_r[...]) + b2_r[...]
        loc_ref[...] = o[:, 0:LAT]
        scale_ref[...] = _softplus(o[:, LAT:2 * LAT]) + 1e-4

    return pl.pallas_call(
        body,
        grid=(N // BN,),
        in_specs=[pl.BlockSpec((BN, LAT), lambda i: (i, 0)),
                  _full((LAT, MLP_HID)), _full((1, MLP_HID)),
                  _full((MLP_HID, 2 * LAT)), _full((1, 2 * LAT))],
        out_specs=[pl.BlockSpec((BN, LAT), lambda i: (i, 0)),
                   pl.BlockSpec((BN, LAT), lambda i: (i, 0))],
        out_shape=[jax.ShapeDtypeStruct((N, LAT), jnp.float32),
                   jax.ShapeDtypeStruct((N, LAT), jnp.float32)],
    )(zgp, L1, b1, L2, b2)


def _vae_inf(zg, zgp, eps, p):
    L1 = p["l1"]["W"]
    b1 = p["l1"]["b"].reshape(1, MLP_HID)
    L2, b2 = p["l2"]["W"], p["l2"]["b"].reshape(1, 2 * LAT)

    def body(zg_ref, zgp_ref, eps_ref, l1_r, b1_r, l2_r, b2_r,
             loc_ref, scale_ref, z_ref):
        cat = jnp.concatenate([zg_ref[...], zgp_ref[...]], axis=1)
        hdn = _silu(_dot(cat, l1_r[...]) + b1_r[...])
        o = _dot(hdn, l2_r[...]) + b2_r[...]
        loc = o[:, 0:LAT]
        scale = _softplus(o[:, LAT:2 * LAT]) + 1e-4
        loc_ref[...] = loc
        scale_ref[...] = scale
        z_ref[...] = loc + scale * eps_ref[...]

    return pl.pallas_call(
        body,
        grid=(N // BN,),
        in_specs=[pl.BlockSpec((BN, LAT), lambda i: (i, 0)),
                  pl.BlockSpec((BN, LAT), lambda i: (i, 0)),
                  pl.BlockSpec((BN, LAT), lambda i: (i, 0)),
                  _full((2 * LAT, MLP_HID)), _full((1, MLP_HID)),
                  _full((MLP_HID, 2 * LAT)), _full((1, 2 * LAT))],
        out_specs=[pl.BlockSpec((BN, LAT), lambda i: (i, 0)),
                   pl.BlockSpec((BN, LAT), lambda i: (i, 0)),
                   pl.BlockSpec((BN, LAT), lambda i: (i, 0))],
        out_shape=[jax.ShapeDtypeStruct((N, LAT), jnp.float32),
                   jax.ShapeDtypeStruct((N, LAT), jnp.float32),
                   jax.ShapeDtypeStruct((N, LAT), jnp.float32)],
    )(zg, zgp, eps, L1, b1, L2, b2)


def _gnn_pass(params, table_x, P, h, ea8, src2d, dst2d, zeros_acc):
    hh = _mlp1(h, params["embed"]["W"], params["embed"]["b"])
    xs, xd = _gather2_kernel(P)(table_x, src2d, dst2d)
    d2 = _d2(xs, xd, P)
    for lp in params["layers"]:
        hs, hd = _gather2_kernel(LAT)(hh, src2d, dst2d)
        m = _edge_mlp(hd, hs, d2, ea8, lp)
        part = _scatter_kernel()(m, dst2d, zeros_acc)
        hh = _node_update(hh, part, lp)
    return _proj(hh, params["out"]["W"], params["out"]["b"])


def kernel(x, h, edge_attr, edge_attr_partial, edge_index, partial_goal_mask,
           enc_goal_params, enc_partial_params, dec_params, inf_params,
           prior_params):
    src = edge_index[0]
    dst = edge_index[1]
    src2d = jnp.pad(src, (0, EPAD - E)).reshape(NIDXROWS, 128)
    dst2d = jnp.pad(dst, (0, EPAD - E),
                    constant_values=N).reshape(NIDXROWS, 128)
    zeros_acc = jnp.zeros((NACC, LAT), jnp.float32)

    x_pad = jnp.pad(x, ((0, 0), (0, H_DIM - 3)))
    xp_pad = partial_goal_mask[:, None] * x_pad
    ea8 = jnp.pad(edge_attr, ((0, EPAD - E), (0, 4)))
    ea8_partial = jnp.pad(edge_attr_partial, ((0, EPAD - E), (0, 4)))

    z_goal = _gnn_pass(enc_goal_params, x_pad, H_DIM, h, ea8,
                       src2d, dst2d, zeros_acc)
    z_goal_partial = _gnn_pass(enc_partial_params, xp_pad, H_DIM, h,
                               ea8_partial, src2d, dst2d, zeros_acc)

    p_loc, p_scale = _vae_prior(z_goal_partial, prior_params)
    eps = jax.random.normal(jax.random.key(42), (N, LAT), jnp.float32)
    q_loc, q_scale, z = _vae_inf(z_goal, z_goal_partial, eps, inf_params)

    dec_table = jnp.concatenate([z, z_goal_partial], axis=-1)
    mu_x_sample = _gnn_pass(dec_params, dec_table, 2 * LAT, h, None,
                            src2d, dst2d, zeros_acc)
    return (mu_x_sample, q_loc, q_scale, p_loc, p_scale)


# speedup vs baseline: 2.2373x; 1.0788x over previous
import functools

import jax
import jax.numpy as jnp
from jax import lax
from jax.experimental import pallas as pl
from jax.experimental.pallas import tpu as pltpu
from jax.experimental.pallas import tpu_sc as plsc

N = 50000
E = 800000
H_DIM = 16
LAT = 32
HID = 64
MLP_HID = 128

EPAD = 819200
NIDXROWS = EPAD // 128
NC = 2
NS = 16
NW = NC * NS
ROWS_PER_W = NIDXROWS // NW
NACC = 50048

BN = 2000
BE = 3200

_mesh = plsc.VectorSubcoreMesh(core_axis_name="c", subcore_axis_name="s")
_sc_params = pltpu.CompilerParams(use_tc_tiling_on_sc=False)


@functools.lru_cache(maxsize=None)
def _gather2_kernel(D):
    G = 8 if D <= LAT else 4
    CH = G * 128
    NG = ROWS_PER_W // G

    @functools.partial(
        pl.kernel,
        mesh=_mesh,
        compiler_params=_sc_params,
        out_type=[jax.ShapeDtypeStruct((EPAD, D), jnp.float32),
                  jax.ShapeDtypeStruct((EPAD, D), jnp.float32)],
        scratch_types=[pltpu.VMEM((2 * ROWS_PER_W, 128), jnp.int32),
                       pltpu.VMEM((CH, D), jnp.float32),
                       pltpu.VMEM((CH, D), jnp.float32),
                       pltpu.SemaphoreType.DMA,
                       pltpu.SemaphoreType.DMA],
    )
    def k(table, src2d, dst2d, o_src, o_dst, idx_all, rows0, rows1,
          sem0, sem1):
        wid = lax.axis_index("s") * NC + lax.axis_index("c")
        row0 = wid * ROWS_PER_W
        e0 = row0 * 128
        pltpu.sync_copy(src2d.at[pl.ds(row0, ROWS_PER_W)],
                        idx_all.at[pl.ds(0, ROWS_PER_W)])
        pltpu.sync_copy(dst2d.at[pl.ds(row0, ROWS_PER_W)],
                        idx_all.at[pl.ds(ROWS_PER_W, ROWS_PER_W)])

        for li, out in ((0, o_src), (1, o_dst)):
            off = li * ROWS_PER_W

            def issue(g, buf, sem):
                for j in range(G):
                    pltpu.async_copy(table.at[idx_all.at[off + g * G + j]],
                                     buf.at[pl.ds(j * 128, 128)], sem)

            def drain(buf, sem):
                pltpu.make_async_copy(table.at[pl.ds(0, CH)], buf, sem).wait()

            def wb(g, buf):
                pltpu.sync_copy(buf, out.at[pl.ds(e0 + g * CH, CH)])

            issue(0, rows0, sem0)
            npairs = (NG - 1) // 2

            def body(t, carry):
                g1 = 2 * t + 1
                issue(g1, rows1, sem1)
                drain(rows0, sem0)
                wb(g1 - 1, rows0)
                issue(g1 + 1, rows0, sem0)
                drain(rows1, sem1)
                wb(g1, rows1)
                return carry

            lax.fori_loop(0, npairs, body, 0)
            if NG % 2 == 1:
                drain(rows0, sem0)
                wb(NG - 1, rows0)
            else:
                issue(NG - 1, rows1, sem1)
                drain(rows0, sem0)
                wb(NG - 2, rows0)
                drain(rows1, sem1)
                wb(NG - 1, rows1)

    return k


SC_G = 2
SC_CH = SC_G * 128
SC_NG = ROWS_PER_W // SC_G


@functools.lru_cache(maxsize=None)
def _scatter_kernel():
    zrows = NACC // NS

    @functools.partial(
        pl.kernel,
        mesh=_mesh,
        compiler_params=_sc_params,
        out_type=jax.ShapeDtypeStruct((NC, NACC, LAT), jnp.float32),
        scratch_types=[pltpu.VMEM((SC_G, 128), jnp.int32),
                       pltpu.VMEM((SC_G, 128), jnp.int32),
                       pltpu.VMEM((SC_CH, LAT), jnp.float32),
                       pltpu.VMEM((SC_CH, LAT), jnp.float32),
                       pltpu.VMEM_SHARED((NACC, LAT), jnp.float32),
                       pltpu.SemaphoreType.DMA,
                       pltpu.SemaphoreType.DMA],
    )
    def k(m, dst2d, zeros_hbm, partial, idx0, idx1, rows0, rows1, acc,
          sem0, sem1):
        c = lax.axis_index("c")
        s = lax.axis_index("s")
        wid = s * NC + c
        row0 = wid * ROWS_PER_W
        e0 = row0 * 128

        pltpu.sync_copy(zeros_hbm.at[pl.ds(s * zrows, zrows)],
                        acc.at[pl.ds(s * zrows, zrows)])
        plsc.subcore_barrier()

        def stage(g, idx, rows):
            pltpu.sync_copy(dst2d.at[pl.ds(row0 + g * SC_G, SC_G)], idx)
            pltpu.sync_copy(m.at[pl.ds(e0 + g * SC_CH, SC_CH)], rows)

        def issue(idx, rows, sem):
            for j in range(SC_G):
                pltpu.async_copy(rows.at[pl.ds(j * 128, 128)],
                                 acc.at[idx.at[j]], sem, add=True)

        def drain(rows, sem):
            pltpu.make_async_copy(rows, acc.at[pl.ds(0, SC_CH)], sem).wait()

        stage(0, idx0, rows0)
        issue(idx0, rows0, sem0)
        npairs = (SC_NG - 2) // 2

        def body(t, carry):
            g1 = 2 * t + 1

            @pl.when(t > 0)
            def _():
                drain(rows1, sem1)

            stage(g1, idx1, rows1)
            issue(idx1, rows1, sem1)
            drain(rows0, sem0)
            stage(g1 + 1, idx0, rows0)
            issue(idx0, rows0, sem0)
            return carry

        lax.fori_loop(0, npairs, body, 0)
        drain(rows1, sem1)
        stage(SC_NG - 1, idx1, rows1)
        issue(idx1, rows1, sem1)
        drain(rows0, sem0)
        drain(rows1, sem1)

        plsc.subcore_barrier()
        pltpu.sync_copy(acc.at[pl.ds(s * zrows, zrows)],
                        partial.at[c, pl.ds(s * zrows, zrows)])

    return k


def _full(shape):
    return pl.BlockSpec(shape, lambda i: tuple(0 for _ in shape))


def _silu(x):
    return x * jax.nn.sigmoid(x)


def _dot(a, b):
    return jnp.dot(a, b, preferred_element_type=jnp.float32)


def _pad_idx(v, fill):
    return jnp.pad(v, (0, EPAD - E),
                   constant_values=fill).reshape(NIDXROWS, 128)


def _mlp1(h, W, b):
    din, dout = W.shape

    def body(h_ref, w_ref, b_ref, o_ref):
        o_ref[...] = _silu(_dot(h_ref[...], w_ref[...]) + b_ref[...])

    return pl.pallas_call(
        body,
        grid=(N // BN,),
        in_specs=[pl.BlockSpec((BN, din), lambda i: (i, 0)),
                  _full((din, dout)), _full((1, dout))],
        out_specs=pl.BlockSpec((BN, dout), lambda i: (i, 0)),
        out_shape=jax.ShapeDtypeStruct((N, dout), jnp.float32),
    )(h, W, b.reshape(1, dout))


def _proj(h, W, b):
    din, dout = W.shape

    def body(h_ref, w_ref, b_ref, o_ref):
        o_ref[...] = _dot(h_ref[...], w_ref[...]) + b_ref[...]

    return pl.pallas_call(
        body,
        grid=(N // BN,),
        in_specs=[pl.BlockSpec((BN, din), lambda i: (i, 0)),
                  _full((din, dout)), _full((1, dout))],
        out_specs=pl.BlockSpec((BN, dout), lambda i: (i, 0)),
        out_shape=jax.ShapeDtypeStruct((N, dout), jnp.float32),
    )(h, W, b.reshape(1, dout))


def _d2(xs, xd, P):
    def body(xs_ref, xd_ref, o_ref):
        r = xs_ref[...] - xd_ref[...]
        o_ref[...] = jnp.sum(r * r, axis=1, keepdims=True)

    return pl.pallas_call(
        body,
        grid=(E // BE,),
        in_specs=[pl.BlockSpec((BE, P), lambda i: (i, 0)),
                  pl.BlockSpec((BE, P), lambda i: (i, 0))],
        out_specs=pl.BlockSpec((BE, 1), lambda i: (i, 0)),
        out_shape=jax.ShapeDtypeStruct((E, 1), jnp.float32),
    )(xs, xd)


def _edge_mlp(hd, hs, d2, ea, lp):
    W1 = jnp.pad(lp["e1"]["W"], ((0, 3), (0, 0)))
    b1 = lp["e1"]["b"].reshape(1, HID)
    W2 = lp["e2"]["W"]
    b2 = lp["e2"]["b"].reshape(1, LAT)
    with_ea = ea is not None

    def body(hd_ref, hs_ref, d2_ref, ea_ref, w1_r, b1_r, w2_r, b2_r, o_ref):
        if with_ea:
            eav = ea_ref[...]
        else:
            eav = jnp.zeros((BE, 4), jnp.float32)
        m_in = jnp.concatenate(
            [hd_ref[...], hs_ref[...], d2_ref[...], eav,
             jnp.zeros((BE, 3), jnp.float32)], axis=1)
        u = _silu(_dot(m_in, w1_r[...]) + b1_r[...])
        o_ref[...] = _silu(_dot(u, w2_r[...]) + b2_r[...])

    args = [hd, hs, d2]
    in_specs = [pl.BlockSpec((BE, LAT), lambda i: (i, 0)),
                pl.BlockSpec((BE, LAT), lambda i: (i, 0)),
                pl.BlockSpec((BE, 1), lambda i: (i, 0))]
    if with_ea:
        args.append(ea)
        in_specs.append(pl.BlockSpec((BE, 4), lambda i: (i, 0)))
    else:
        args.append(d2)
        in_specs.append(pl.BlockSpec((BE, 1), lambda i: (i, 0)))
    args += [W1, b1, W2, b2]
    in_specs += [_full((2 * LAT + 8, HID)), _full((1, HID)),
                 _full((HID, LAT)), _full((1, LAT))]

    return pl.pallas_call(
        body,
        grid=(E // BE,),
        in_specs=in_specs,
        out_specs=pl.BlockSpec((BE, LAT), lambda i: (i, 0)),
        out_shape=jax.ShapeDtypeStruct((EPAD, LAT), jnp.float32),
    )(*args)


def _node_update(hh, part, lp):
    Wh1 = lp["h1"]["W"]
    bh1 = lp["h1"]["b"].reshape(1, HID)
    Wh2 = lp["h2"]["W"]
    bh2 = lp["h2"]["b"].reshape(1, LAT)
    p0 = part[0]
    p1 = part[1]

    def body(hh_ref, p0_ref, p1_ref, w1_r, b1_r, w2_r, b2_r, o_ref):
        agg = p0_ref[...] + p1_ref[...]
        cat = jnp.concatenate([hh_ref[...], agg], axis=1)
        u = _silu(_dot(cat, w1_r[...]) + b1_r[...])
        o_ref[...] = hh_ref[...] + _dot(u, w2_r[...]) + b2_r[...]

    return pl.pallas_call(
        body,
        grid=(N // BN,),
        in_specs=[pl.BlockSpec((BN, LAT), lambda i: (i, 0)),
                  pl.BlockSpec((BN, LAT), lambda i: (i, 0)),
                  pl.BlockSpec((BN, LAT), lambda i: (i, 0)),
                  _full((2 * LAT, HID)), _full((1, HID)),
                  _full((HID, LAT)), _full((1, LAT))],
        out_specs=pl.BlockSpec((BN, LAT), lambda i: (i, 0)),
        out_shape=jax.ShapeDtypeStruct((N, LAT), jnp.float32),
    )(hh, p0, p1, Wh1, bh1, Wh2, bh2)


def _softplus(x):
    return jnp.maximum(x, 0.0) + jnp.log1p(jnp.exp(-jnp.abs(x)))


def _vae_prior(zgp, p):
    L1, b1 = p["l1"]["W"], p["l1"]["b"].reshape(1, MLP_HID)
    L2, b2 = p["l2"]["W"], p["l2"]["b"].reshape(1, 2 * LAT)

    def body(z_ref, l1_r, b1_r, l2_r, b2_r, loc_ref, scale_ref):
        hdn = _silu(_dot(z_ref[...], l1_r[...]) + b1_r[...])
        o = _dot(hdn, l2_r[...]) + b2_r[...]
        loc_ref[...] = o[:, 0:LAT]
        scale_ref[...] = _softplus(o[:, LAT:2 * LAT]) + 1e-4

    return pl.pallas_call(
        body,
        grid=(N // BN,),
        in_specs=[pl.BlockSpec((BN, LAT), lambda i: (i, 0)),
                  _full((LAT, MLP_HID)), _full((1, MLP_HID)),
                  _full((MLP_HID, 2 * LAT)), _full((1, 2 * LAT))],
        out_specs=[pl.BlockSpec((BN, LAT), lambda i: (i, 0)),
                   pl.BlockSpec((BN, LAT), lambda i: (i, 0))],
        out_shape=[jax.ShapeDtypeStruct((N, LAT), jnp.float32),
                   jax.ShapeDtypeStruct((N, LAT), jnp.float32)],
    )(zgp, L1, b1, L2, b2)


def _vae_inf(zg, zgp, eps, p):
    L1 = p["l1"]["W"]
    b1 = p["l1"]["b"].reshape(1, MLP_HID)
    L2, b2 = p["l2"]["W"], p["l2"]["b"].reshape(1, 2 * LAT)

    def body(zg_ref, zgp_ref, eps_ref, l1_r, b1_r, l2_r, b2_r,
             loc_ref, scale_ref, z_ref):
        cat = jnp.concatenate([zg_ref[...], zgp_ref[...]], axis=1)
        hdn = _silu(_dot(cat, l1_r[...]) + b1_r[...])
        o = _dot(hdn, l2_r[...]) + b2_r[...]
        loc = o[:, 0:LAT]
        scale = _softplus(o[:, LAT:2 * LAT]) + 1e-4
        loc_ref[...] = loc
        scale_ref[...] = scale
        z_ref[...] = loc + scale * eps_ref[...]

    return pl.pallas_call(
        body,
        grid=(N // BN,),
        in_specs=[pl.BlockSpec((BN, LAT), lambda i: (i, 0)),
                  pl.BlockSpec((BN, LAT), lambda i: (i, 0)),
                  pl.BlockSpec((BN, LAT), lambda i: (i, 0)),
                  _full((2 * LAT, MLP_HID)), _full((1, MLP_HID)),
                  _full((MLP_HID, 2 * LAT)), _full((1, 2 * LAT))],
        out_specs=[pl.BlockSpec((BN, LAT), lambda i: (i, 0)),
                   pl.BlockSpec((BN, LAT), lambda i: (i, 0)),
                   pl.BlockSpec((BN, LAT), lambda i: (i, 0))],
        out_shape=[jax.ShapeDtypeStruct((N, LAT), jnp.float32),
                   jax.ShapeDtypeStruct((N, LAT), jnp.float32),
                   jax.ShapeDtypeStruct((N, LAT), jnp.float32)],
    )(zg, zgp, eps, L1, b1, L2, b2)


def _gnn_pass(params, table_x, P, h, ea, src2d, dst2d, zeros_acc):
    hh = _mlp1(h, params["embed"]["W"], params["embed"]["b"])
    xs, xd = _gather2_kernel(P)(table_x, src2d, dst2d)
    d2 = _d2(xs, xd, P)
    for lp in params["layers"]:
        hs, hd = _gather2_kernel(LAT)(hh, src2d, dst2d)
        m = _edge_mlp(hd, hs, d2, ea, lp)
        part = _scatter_kernel()(m, dst2d, zeros_acc)
        hh = _node_update(hh, part, lp)
    return _proj(hh, params["out"]["W"], params["out"]["b"])


def kernel(x, h, edge_attr, edge_attr_partial, edge_index, partial_goal_mask,
           enc_goal_params, enc_partial_params, dec_params, inf_params,
           prior_params):
    src = edge_index[0]
    dst = edge_index[1]
    src2d = _pad_idx(src, 0)
    dst2d = _pad_idx(dst, N)
    zeros_acc = jnp.zeros((NACC, LAT), jnp.float32)

    x_pad = jnp.pad(x, ((0, 0), (0, H_DIM - 3)))
    xp_pad = partial_goal_mask[:, None] * x_pad

    z_goal = _gnn_pass(enc_goal_params, x_pad, H_DIM, h, edge_attr,
                       src2d, dst2d, zeros_acc)
    z_goal_partial = _gnn_pass(enc_partial_params, xp_pad, H_DIM, h,
                               edge_attr_partial, src2d, dst2d, zeros_acc)

    p_loc, p_scale = _vae_prior(z_goal_partial, prior_params)
    eps = jax.random.normal(jax.random.key(42), (N, LAT), jnp.float32)
    q_loc, q_scale, z = _vae_inf(z_goal, z_goal_partial, eps, inf_params)

    dec_table = jnp.concatenate([z, z_goal_partial], axis=-1)
    mu_x_sample = _gnn_pass(dec_params, dec_table, 2 * LAT, h, None,
                            src2d, dst2d, zeros_acc)
    return (mu_x_sample, q_loc, q_scale, p_loc, p_scale)


# speedup vs baseline: 2.6754x; 1.1958x over previous
import functools

import jax
import jax.numpy as jnp
from jax import lax
from jax.experimental import pallas as pl
from jax.experimental.pallas import tpu as pltpu
from jax.experimental.pallas import tpu_sc as plsc

N = 50000
E = 800000
H_DIM = 16
LAT = 32
HID = 64
MLP_HID = 128

EPAD = 819200
NIDXROWS = EPAD // 128
NC = 2
NS = 16
NW = NC * NS
ROWS_PER_W = NIDXROWS // NW
NACC = 50048

BN = 2000
BE = 3200

_mesh = plsc.VectorSubcoreMesh(core_axis_name="c", subcore_axis_name="s")
_sc_params = pltpu.CompilerParams(use_tc_tiling_on_sc=False)


@functools.lru_cache(maxsize=None)
def _gather2_kernel(D):
    G = 2 if D >= LAT else 4
    CH = G * 128
    NG = ROWS_PER_W // G
    trows = N // NS

    @functools.partial(
        pl.kernel,
        mesh=_mesh,
        compiler_params=_sc_params,
        out_type=[jax.ShapeDtypeStruct((EPAD, D), jnp.float32),
                  jax.ShapeDtypeStruct((EPAD, D), jnp.float32)],
        scratch_types=[pltpu.VMEM((G, 128), jnp.int32),
                       pltpu.VMEM((G, 128), jnp.int32),
                       pltpu.VMEM((CH, D), jnp.float32),
                       pltpu.VMEM((CH, D), jnp.float32),
                       pltpu.VMEM_SHARED((NACC, D), jnp.float32),
                       pltpu.SemaphoreType.DMA,
                       pltpu.SemaphoreType.DMA],
    )
    def k(table, src2d, dst2d, o_src, o_dst, idx0, idx1, rows0, rows1,
          tspm, sem0, sem1):
        s = lax.axis_index("s")
        wid = s * NC + lax.axis_index("c")
        row0 = wid * ROWS_PER_W
        e0 = row0 * 128
        pltpu.sync_copy(table.at[pl.ds(s * trows, trows)],
                        tspm.at[pl.ds(s * trows, trows)])
        plsc.subcore_barrier()

        for li, out in ((0, o_src), (1, o_dst)):
            ind = src2d if li == 0 else dst2d

            def stage(g, idx):
                pltpu.sync_copy(ind.at[pl.ds(row0 + g * G, G)], idx)

            def issue(idx, rows, sem):
                for j in range(G):
                    pltpu.async_copy(tspm.at[idx.at[j]],
                                     rows.at[pl.ds(j * 128, 128)], sem)

            def drain(rows, sem):
                pltpu.make_async_copy(table.at[pl.ds(0, CH)], rows, sem).wait()

            def wb(g, rows):
                pltpu.sync_copy(rows, out.at[pl.ds(e0 + g * CH, CH)])

            stage(0, idx0)
            issue(idx0, rows0, sem0)
            npairs = (NG - 2) // 2

            def body(t, carry):
                g1 = 2 * t + 1
                stage(g1, idx1)
                issue(idx1, rows1, sem1)
                drain(rows0, sem0)
                wb(g1 - 1, rows0)
                stage(g1 + 1, idx0)
                issue(idx0, rows0, sem0)
                drain(rows1, sem1)
                wb(g1, rows1)
                return carry

            lax.fori_loop(0, npairs, body, 0)
            stage(NG - 1, idx1)
            issue(idx1, rows1, sem1)
            drain(rows0, sem0)
            wb(NG - 2, rows0)
            drain(rows1, sem1)
            wb(NG - 1, rows1)

    return k


SC_G = 2
SC_CH = SC_G * 128
SC_NG = ROWS_PER_W // SC_G


@functools.lru_cache(maxsize=None)
def _scatter_kernel():
    zrows = NACC // NS

    @functools.partial(
        pl.kernel,
        mesh=_mesh,
        compiler_params=_sc_params,
        out_type=jax.ShapeDtypeStruct((NC, NACC, LAT), jnp.float32),
        scratch_types=[pltpu.VMEM((SC_G, 128), jnp.int32),
                       pltpu.VMEM((SC_G, 128), jnp.int32),
                       pltpu.VMEM((SC_CH, LAT), jnp.float32),
                       pltpu.VMEM((SC_CH, LAT), jnp.float32),
                       pltpu.VMEM_SHARED((NACC, LAT), jnp.float32),
                       pltpu.SemaphoreType.DMA,
                       pltpu.SemaphoreType.DMA],
    )
    def k(m, dst2d, zeros_hbm, partial, idx0, idx1, rows0, rows1, acc,
          sem0, sem1):
        c = lax.axis_index("c")
        s = lax.axis_index("s")
        wid = s * NC + c
        row0 = wid * ROWS_PER_W
        e0 = row0 * 128

        pltpu.sync_copy(zeros_hbm.at[pl.ds(s * zrows, zrows)],
                        acc.at[pl.ds(s * zrows, zrows)])
        plsc.subcore_barrier()

        def stage(g, idx, rows):
            pltpu.sync_copy(dst2d.at[pl.ds(row0 + g * SC_G, SC_G)], idx)
            pltpu.sync_copy(m.at[pl.ds(e0 + g * SC_CH, SC_CH)], rows)

        def issue(idx, rows, sem):
            for j in range(SC_G):
                pltpu.async_copy(rows.at[pl.ds(j * 128, 128)],
                                 acc.at[idx.at[j]], sem, add=True)

        def drain(rows, sem):
            pltpu.make_async_copy(rows, acc.at[pl.ds(0, SC_CH)], sem).wait()

        stage(0, idx0, rows0)
        issue(idx0, rows0, sem0)
        npairs = (SC_NG - 2) // 2

        def body(t, carry):
            g1 = 2 * t + 1

            @pl.when(t > 0)
            def _():
                drain(rows1, sem1)

            stage(g1, idx1, rows1)
            issue(idx1, rows1, sem1)
            drain(rows0, sem0)
            stage(g1 + 1, idx0, rows0)
            issue(idx0, rows0, sem0)
            return carry

        lax.fori_loop(0, npairs, body, 0)
        drain(rows1, sem1)
        stage(SC_NG - 1, idx1, rows1)
        issue(idx1, rows1, sem1)
        drain(rows0, sem0)
        drain(rows1, sem1)

        plsc.subcore_barrier()
        pltpu.sync_copy(acc.at[pl.ds(s * zrows, zrows)],
                        partial.at[c, pl.ds(s * zrows, zrows)])

    return k


def _full(shape):
    return pl.BlockSpec(shape, lambda i: tuple(0 for _ in shape))


def _silu(x):
    return x * jax.nn.sigmoid(x)


def _dot(a, b):
    return jnp.dot(a, b, preferred_element_type=jnp.float32)


def _pad_idx(v, fill):
    return jnp.pad(v, (0, EPAD - E),
                   constant_values=fill).reshape(NIDXROWS, 128)


def _mlp1(h, W, b):
    din, dout = W.shape

    def body(h_ref, w_ref, b_ref, o_ref):
        o_ref[...] = _silu(_dot(h_ref[...], w_ref[...]) + b_ref[...])

    return pl.pallas_call(
        body,
        grid=(N // BN,),
        in_specs=[pl.BlockSpec((BN, din), lambda i: (i, 0)),
                  _full((din, dout)), _full((1, dout))],
        out_specs=pl.BlockSpec((BN, dout), lambda i: (i, 0)),
        out_shape=jax.ShapeDtypeStruct((N, dout), jnp.float32),
    )(h, W, b.reshape(1, dout))


def _proj(h, W, b):
    din, dout = W.shape

    def body(h_ref, w_ref, b_ref, o_ref):
        o_ref[...] = _dot(h_ref[...], w_ref[...]) + b_ref[...]

    return pl.pallas_call(
        body,
        grid=(N // BN,),
        in_specs=[pl.BlockSpec((BN, din), lambda i: (i, 0)),
                  _full((din, dout)), _full((1, dout))],
        out_specs=pl.BlockSpec((BN, dout), lambda i: (i, 0)),
        out_shape=jax.ShapeDtypeStruct((N, dout), jnp.float32),
    )(h, W, b.reshape(1, dout))


def _d2(xs, xd, P):
    def body(xs_ref, xd_ref, o_ref):
        r = xs_ref[...] - xd_ref[...]
        o_ref[...] = jnp.sum(r * r, axis=1, keepdims=True)

    return pl.pallas_call(
        body,
        grid=(E // BE,),
        in_specs=[pl.BlockSpec((BE, P), lambda i: (i, 0)),
                  pl.BlockSpec((BE, P), lambda i: (i, 0))],
        out_specs=pl.BlockSpec((BE, 1), lambda i: (i, 0)),
        out_shape=jax.ShapeDtypeStruct((E, 1), jnp.float32),
    )(xs, xd)


def _d2_two(xs1, xd1, xs2, xd2):
    def body(a_ref, b_ref, c_ref, d_ref, o_ref):
        r1 = a_ref[...] - b_ref[...]
        r2 = c_ref[...] - d_ref[...]
        o_ref[...] = (jnp.sum(r1 * r1, axis=1, keepdims=True)
                      + jnp.sum(r2 * r2, axis=1, keepdims=True))

    blk = pl.BlockSpec((BE, LAT), lambda i: (i, 0))
    return pl.pallas_call(
        body,
        grid=(E // BE,),
        in_specs=[blk, blk, blk, blk],
        out_specs=pl.BlockSpec((BE, 1), lambda i: (i, 0)),
        out_shape=jax.ShapeDtypeStruct((E, 1), jnp.float32),
    )(xs1, xd1, xs2, xd2)


def _edge_mlp(hd, hs, d2, ea, lp):
    W1 = jnp.pad(lp["e1"]["W"], ((0, 3), (0, 0)))
    b1 = lp["e1"]["b"].reshape(1, HID)
    W2 = lp["e2"]["W"]
    b2 = lp["e2"]["b"].reshape(1, LAT)
    with_ea = ea is not None

    def body(hd_ref, hs_ref, d2_ref, ea_ref, w1_r, b1_r, w2_r, b2_r, o_ref):
        if with_ea:
            eav = ea_ref[...]
        else:
            eav = jnp.zeros((BE, 4), jnp.float32)
        m_in = jnp.concatenate(
            [hd_ref[...], hs_ref[...], d2_ref[...], eav,
             jnp.zeros((BE, 3), jnp.float32)], axis=1)
        u = _silu(_dot(m_in, w1_r[...]) + b1_r[...])
        o_ref[...] = _silu(_dot(u, w2_r[...]) + b2_r[...])

    args = [hd, hs, d2]
    in_specs = [pl.BlockSpec((BE, LAT), lambda i: (i, 0)),
                pl.BlockSpec((BE, LAT), lambda i: (i, 0)),
                pl.BlockSpec((BE, 1), lambda i: (i, 0))]
    if with_ea:
        args.append(ea)
        in_specs.append(pl.BlockSpec((BE, 4), lambda i: (i, 0)))
    else:
        args.append(d2)
        in_specs.append(pl.BlockSpec((BE, 1), lambda i: (i, 0)))
    args += [W1, b1, W2, b2]
    in_specs += [_full((2 * LAT + 8, HID)), _full((1, HID)),
                 _full((HID, LAT)), _full((1, LAT))]

    return pl.pallas_call(
        body,
        grid=(E // BE,),
        in_specs=in_specs,
        out_specs=pl.BlockSpec((BE, LAT), lambda i: (i, 0)),
        out_shape=jax.ShapeDtypeStruct((EPAD, LAT), jnp.float32),
    )(*args)


def _node_update(hh, part, lp):
    Wh1 = lp["h1"]["W"]
    bh1 = lp["h1"]["b"].reshape(1, HID)
    Wh2 = lp["h2"]["W"]
    bh2 = lp["h2"]["b"].reshape(1, LAT)
    p0 = part[0]
    p1 = part[1]

    def body(hh_ref, p0_ref, p1_ref, w1_r, b1_r, w2_r, b2_r, o_ref):
        agg = p0_ref[...] + p1_ref[...]
        cat = jnp.concatenate([hh_ref[...], agg], axis=1)
        u = _silu(_dot(cat, w1_r[...]) + b1_r[...])
        o_ref[...] = hh_ref[...] + _dot(u, w2_r[...]) + b2_r[...]

    return pl.pallas_call(
        body,
        grid=(N // BN,),
        in_specs=[pl.BlockSpec((BN, LAT), lambda i: (i, 0)),
                  pl.BlockSpec((BN, LAT), lambda i: (i, 0)),
                  pl.BlockSpec((BN, LAT), lambda i: (i, 0)),
                  _full((2 * LAT, HID)), _full((1, HID)),
                  _full((HID, LAT)), _full((1, LAT))],
        out_specs=pl.BlockSpec((BN, LAT), lambda i: (i, 0)),
        out_shape=jax.ShapeDtypeStruct((N, LAT), jnp.float32),
    )(hh, p0, p1, Wh1, bh1, Wh2, bh2)


def _softplus(x):
    return jnp.maximum(x, 0.0) + jnp.log1p(jnp.exp(-jnp.abs(x)))


def _vae_prior(zgp, p):
    L1, b1 = p["l1"]["W"], p["l1"]["b"].reshape(1, MLP_HID)
    L2, b2 = p["l2"]["W"], p["l2"]["b"].reshape(1, 2 * LAT)

    def body(z_ref, l1_r, b1_r, l2_r, b2_r, loc_ref, scale_ref):
        hdn = _silu(_dot(z_ref[...], l1_r[...]) + b1_r[...])
        o = _dot(hdn, l2_r[...]) + b2_r[...]
        loc_ref[...] = o[:, 0:LAT]
        scale_ref[...] = _softplus(o[:, LAT:2 * LAT]) + 1e-4

    return pl.pallas_call(
        body,
        grid=(N // BN,),
        in_specs=[pl.BlockSpec((BN, LAT), lambda i: (i, 0)),
                  _full((LAT, MLP_HID)), _full((1, MLP_HID)),
                  _full((MLP_HID, 2 * LAT)), _full((1, 2 * LAT))],
        out_specs=[pl.BlockSpec((BN, LAT), lambda i: (i, 0)),
                   pl.BlockSpec((BN, LAT), lambda i: (i, 0))],
        out_shape=[jax.ShapeDtypeStruct((N, LAT), jnp.float32),
                   jax.ShapeDtypeStruct((N, LAT), jnp.float32)],
    )(zgp, L1, b1, L2, b2)


def _vae_inf(zg, zgp, eps, p):
    L1 = p["l1"]["W"]
    b1 = p["l1"]["b"].reshape(1, MLP_HID)
    L2, b2 = p["l2"]["W"], p["l2"]["b"].reshape(1, 2 * LAT)

    def body(zg_ref, zgp_ref, eps_ref, l1_r, b1_r, l2_r, b2_r,
             loc_ref, scale_ref, z_ref):
        cat = jnp.concatenate([zg_ref[...], zgp_ref[...]], axis=1)
        hdn = _silu(_dot(cat, l1_r[...]) + b1_r[...])
        o = _dot(hdn, l2_r[...]) + b2_r[...]
        loc = o[:, 0:LAT]
        scale = _softplus(o[:, LAT:2 * LAT]) + 1e-4
        loc_ref[...] = loc
        scale_ref[...] = scale
        z_ref[...] = loc + scale * eps_ref[...]

    return pl.pallas_call(
        body,
        grid=(N // BN,),
        in_specs=[pl.BlockSpec((BN, LAT), lambda i: (i, 0)),
                  pl.BlockSpec((BN, LAT), lambda i: (i, 0)),
                  pl.BlockSpec((BN, LAT), lambda i: (i, 0)),
                  _full((2 * LAT, MLP_HID)), _full((1, MLP_HID)),
                  _full((MLP_HID, 2 * LAT)), _full((1, 2 * LAT))],
        out_specs=[pl.BlockSpec((BN, LAT), lambda i: (i, 0)),
                   pl.BlockSpec((BN, LAT), lambda i: (i, 0)),
                   pl.BlockSpec((BN, LAT), lambda i: (i, 0))],
        out_shape=[jax.ShapeDtypeStruct((N, LAT), jnp.float32),
                   jax.ShapeDtypeStruct((N, LAT), jnp.float32),
                   jax.ShapeDtypeStruct((N, LAT), jnp.float32)],
    )(zg, zgp, eps, L1, b1, L2, b2)


def _gnn_pass(params, tables, h, ea, src2d, dst2d, zeros_acc):
    hh = _mlp1(h, params["embed"]["W"], params["embed"]["b"])
    if len(tables) == 1:
        t0 = tables[0]
        xs, xd = _gather2_kernel(t0.shape[1])(t0, src2d, dst2d)
        d2 = _d2(xs, xd, t0.shape[1])
    else:
        xs1, xd1 = _gather2_kernel(LAT)(tables[0], src2d, dst2d)
        xs2, xd2 = _gather2_kernel(LAT)(tables[1], src2d, dst2d)
        d2 = _d2_two(xs1, xd1, xs2, xd2)
    for lp in params["layers"]:
        hs, hd = _gather2_kernel(LAT)(hh, src2d, dst2d)
        m = _edge_mlp(hd, hs, d2, ea, lp)
        part = _scatter_kernel()(m, dst2d, zeros_acc)
        hh = _node_update(hh, part, lp)
    return _proj(hh, params["out"]["W"], params["out"]["b"])


def kernel(x, h, edge_attr, edge_attr_partial, edge_index, partial_goal_mask,
           enc_goal_params, enc_partial_params, dec_params, inf_params,
           prior_params):
    src = edge_index[0]
    dst = edge_index[1]
    src2d = _pad_idx(src, 0)
    dst2d = _pad_idx(dst, N)
    zeros_acc = jnp.zeros((NACC, LAT), jnp.float32)

    x_pad = jnp.pad(x, ((0, 0), (0, H_DIM - 3)))
    xp_pad = partial_goal_mask[:, None] * x_pad

    z_goal = _gnn_pass(enc_goal_params, (x_pad,), h, edge_attr,
                       src2d, dst2d, zeros_acc)
    z_goal_partial = _gnn_pass(enc_partial_params, (xp_pad,), h,
                               edge_attr_partial, src2d, dst2d, zeros_acc)

    p_loc, p_scale = _vae_prior(z_goal_partial, prior_params)
    eps = jax.random.normal(jax.random.key(42), (N, LAT), jnp.float32)
    q_loc, q_scale, z = _vae_inf(z_goal, z_goal_partial, eps, inf_params)

    mu_x_sample = _gnn_pass(dec_params, (z, z_goal_partial), h, None,
                            src2d, dst2d, zeros_acc)
    return (mu_x_sample, q_loc, q_scale, p_loc, p_scale)


# speedup vs baseline: 6.1483x; 2.2981x over previous
import functools

import jax
import jax.numpy as jnp
from jax import lax
from jax.experimental import pallas as pl
from jax.experimental.pallas import tpu as pltpu
from jax.experimental.pallas import tpu_sc as plsc

N = 50000
E = 800000
H_DIM = 16
LAT = 32
HID = 64
MLP_HID = 128

EPAD = 819200
NIDXROWS = EPAD // 128
NC = 2
NS = 16
NW = NC * NS
ROWS_PER_W = NIDXROWS // NW
NACC = 50048

BN = 2000
BE = 3200
E4 = E // 4
EPAD4 = EPAD // 4
BE4 = 2000

_mesh = plsc.VectorSubcoreMesh(core_axis_name="c", subcore_axis_name="s")
_sc_params = pltpu.CompilerParams(use_tc_tiling_on_sc=False)


@functools.lru_cache(maxsize=None)
def _gather2_kernel(D=LAT):
    G = 2 if D >= LAT else 4
    CH = G * 128
    NG = ROWS_PER_W // G
    trows = N // NS

    @functools.partial(
        pl.kernel,
        mesh=_mesh,
        compiler_params=_sc_params,
        out_type=[jax.ShapeDtypeStruct((EPAD, D), jnp.float32),
                  jax.ShapeDtypeStruct((EPAD, D), jnp.float32)],
        scratch_types=[pltpu.VMEM((G, 128), jnp.int32),
                       pltpu.VMEM((G, 128), jnp.int32),
                       pltpu.VMEM((CH, D), jnp.float32),
                       pltpu.VMEM((CH, D), jnp.float32),
                       pltpu.VMEM_SHARED((NACC, D), jnp.float32),
                       pltpu.SemaphoreType.DMA,
                       pltpu.SemaphoreType.DMA],
    )
    def k(table, src2d, dst2d, o_src, o_dst, idx0, idx1, rows0, rows1,
          tspm, sem0, sem1):
        s = lax.axis_index("s")
        wid = s * NC + lax.axis_index("c")
        row0 = wid * ROWS_PER_W
        e0 = row0 * 128
        pltpu.sync_copy(table.at[pl.ds(s * trows, trows)],
                        tspm.at[pl.ds(s * trows, trows)])
        plsc.subcore_barrier()

        for li, out in ((0, o_src), (1, o_dst)):
            ind = src2d if li == 0 else dst2d

            def stage(g, idx):
                pltpu.sync_copy(ind.at[pl.ds(row0 + g * G, G)], idx)

            def issue(idx, rows, sem):
                for j in range(G):
                    pltpu.async_copy(tspm.at[idx.at[j]],
                                     rows.at[pl.ds(j * 128, 128)], sem)

            def drain(rows, sem):
                pltpu.make_async_copy(table.at[pl.ds(0, CH)], rows, sem).wait()

            def wb(g, rows):
                pltpu.sync_copy(rows, out.at[pl.ds(e0 + g * CH, CH)])

            stage(0, idx0)
            issue(idx0, rows0, sem0)
            npairs = (NG - 2) // 2

            def body(t, carry):
                g1 = 2 * t + 1
                stage(g1, idx1)
                issue(idx1, rows1, sem1)
                drain(rows0, sem0)
                wb(g1 - 1, rows0)
                stage(g1 + 1, idx0)
                issue(idx0, rows0, sem0)
                drain(rows1, sem1)
                wb(g1, rows1)
                return carry

            lax.fori_loop(0, npairs, body, 0)
            stage(NG - 1, idx1)
            issue(idx1, rows1, sem1)
            drain(rows0, sem0)
            wb(NG - 2, rows0)
            drain(rows1, sem1)
            wb(NG - 1, rows1)

    return k


SC_G = 2
SC_CH = SC_G * 128
SC_NG = ROWS_PER_W // SC_G


@functools.lru_cache(maxsize=None)
def _scatter_kernel():
    zrows = NACC // NS

    @functools.partial(
        pl.kernel,
        mesh=_mesh,
        compiler_params=_sc_params,
        out_type=jax.ShapeDtypeStruct((NC, NACC, LAT), jnp.float32),
        scratch_types=[pltpu.VMEM((SC_G, 128), jnp.int32),
                       pltpu.VMEM((SC_G, 128), jnp.int32),
                       pltpu.VMEM((SC_CH, LAT), jnp.float32),
                       pltpu.VMEM((SC_CH, LAT), jnp.float32),
                       pltpu.VMEM_SHARED((NACC, LAT), jnp.float32),
                       pltpu.SemaphoreType.DMA,
                       pltpu.SemaphoreType.DMA],
    )
    def k(m, dst2d, zeros_hbm, partial, idx0, idx1, rows0, rows1, acc,
          sem0, sem1):
        c = lax.axis_index("c")
        s = lax.axis_index("s")
        wid = s * NC + c
        row0 = wid * ROWS_PER_W
        e0 = row0 * 128

        pltpu.sync_copy(zeros_hbm.at[pl.ds(s * zrows, zrows)],
                        acc.at[pl.ds(s * zrows, zrows)])
        plsc.subcore_barrier()

        def stage(g, idx, rows):
            pltpu.sync_copy(dst2d.at[pl.ds(row0 + g * SC_G, SC_G)], idx)
            pltpu.sync_copy(m.at[pl.ds(e0 + g * SC_CH, SC_CH)], rows)

        def issue(idx, rows, sem):
            for j in range(SC_G):
                pltpu.async_copy(rows.at[pl.ds(j * 128, 128)],
                                 acc.at[idx.at[j]], sem, add=True)

        def drain(rows, sem):
            pltpu.make_async_copy(rows, acc.at[pl.ds(0, SC_CH)], sem).wait()

        stage(0, idx0, rows0)
        issue(idx0, rows0, sem0)
        npairs = (SC_NG - 2) // 2

        def body(t, carry):
            g1 = 2 * t + 1

            @pl.when(t > 0)
            def _():
                drain(rows1, sem1)

            stage(g1, idx1, rows1)
            issue(idx1, rows1, sem1)
            drain(rows0, sem0)
            stage(g1 + 1, idx0, rows0)
            issue(idx0, rows0, sem0)
            return carry

        lax.fori_loop(0, npairs, body, 0)
        drain(rows1, sem1)
        stage(SC_NG - 1, idx1, rows1)
        issue(idx1, rows1, sem1)
        drain(rows0, sem0)
        drain(rows1, sem1)

        plsc.subcore_barrier()
        pltpu.sync_copy(acc.at[pl.ds(s * zrows, zrows)],
                        partial.at[c, pl.ds(s * zrows, zrows)])

    return k


def _full(shape):
    return pl.BlockSpec(shape, lambda i: tuple(0 for _ in shape))


def _silu(x):
    return x * jax.nn.sigmoid(x)


def _dot(a, b):
    return jnp.dot(a, b, preferred_element_type=jnp.float32)


def _dotx(a, b):
    return jnp.dot(a, b, preferred_element_type=jnp.float32,
                   precision=jax.lax.Precision.HIGHEST)


def _pad_idx(v, fill):
    return jnp.pad(v, (0, EPAD - E),
                   constant_values=fill).reshape(NIDXROWS, 128)


def _mlp1(h, W, b):
    din, dout = W.shape

    def body(h_ref, w_ref, b_ref, o_ref):
        o_ref[...] = _silu(_dot(h_ref[...], w_ref[...]) + b_ref[...])

    return pl.pallas_call(
        body,
        grid=(N // BN,),
        in_specs=[pl.BlockSpec((BN, din), lambda i: (i, 0)),
                  _full((din, dout)), _full((1, dout))],
        out_specs=pl.BlockSpec((BN, dout), lambda i: (i, 0)),
        out_shape=jax.ShapeDtypeStruct((N, dout), jnp.float32),
    )(h, W, b.reshape(1, dout))


def _proj(h, W, b):
    din, dout = W.shape

    def body(h_ref, w_ref, b_ref, o_ref):
        o_ref[...] = _dot(h_ref[...], w_ref[...]) + b_ref[...]

    return pl.pallas_call(
        body,
        grid=(N // BN,),
        in_specs=[pl.BlockSpec((BN, din), lambda i: (i, 0)),
                  _full((din, dout)), _full((1, dout))],
        out_specs=pl.BlockSpec((BN, dout), lambda i: (i, 0)),
        out_shape=jax.ShapeDtypeStruct((N, dout), jnp.float32),
    )(h, W, b.reshape(1, dout))


def _aux_blockdiag_ones():
    out = jnp.zeros((128, 4), jnp.float32)
    for j in range(4):
        out = out.at[32 * j:32 * (j + 1), j].set(1.0)
    return out


def _aux_perms():
    s1 = jnp.zeros((4, 32), jnp.float32)
    s2 = jnp.zeros((16, 32), jnp.float32)
    for j in range(4):
        s1 = s1.at[j, 8 * j].set(1.0)
        for kk in range(4):
            s2 = s2.at[4 * j + kk, 8 * j + 1 + kk].set(1.0)
    return s1, s2


def _d2aux(pairs, ea_p):
    onesb = _aux_blockdiag_ones()
    s1, s2 = _aux_perms()
    with_ea = ea_p is not None
    npairs = len(pairs)

    def body(*refs):
        o_ref = refs[-1]
        d2m = None
        for i in range(npairs):
            r = refs[2 * i][...] - refs[2 * i + 1][...]
            t = _dotx(r * r, refs[-4][...])
            d2m = t if d2m is None else d2m + t
        aux32 = _dotx(d2m, refs[-3][...])
        if with_ea:
            aux32 = aux32 + _dotx(refs[2 * npairs][...], refs[-2][...])
        o_ref[...] = jnp.concatenate(
            [aux32, jnp.zeros((BE4, 96), jnp.float32)], axis=1)

    blk = pl.BlockSpec((BE4, 128), lambda i: (i, 0))
    args = []
    in_specs = []
    for xs_p, xd_p in pairs:
        args += [xs_p, xd_p]
        in_specs += [blk, blk]
    if with_ea:
        args.append(ea_p)
        in_specs.append(pl.BlockSpec((BE4, 16), lambda i: (i, 0)))
    args += [onesb, s1, s2]
    in_specs += [_full((128, 4)), _full((4, 32)), _full((16, 32))]

    return pl.pallas_call(
        body,
        grid=(E4 // BE4,),
        in_specs=in_specs,
        out_specs=blk,
        out_shape=jax.ShapeDtypeStruct((E4, 128), jnp.float32),
    )(*args)


def _edge_mlp(hd_p, hs_p, aux_p, lp):
    W1 = lp["e1"]["W"]
    W2 = lp["e2"]["W"]
    Wb = jnp.zeros((384, 4 * HID), jnp.float32)
    W2b = jnp.zeros((4 * HID, 4 * LAT), jnp.float32)
    for j in range(4):
        c = slice(HID * j, HID * (j + 1))
        Wb = Wb.at[32 * j:32 * (j + 1), c].set(W1[0:32])
        Wb = Wb.at[128 + 32 * j:128 + 32 * (j + 1), c].set(W1[32:64])
        Wb = Wb.at[256 + 8 * j, c].set(W1[64])
        Wb = Wb.at[256 + 8 * j + 1:256 + 8 * j + 5, c].set(W1[65:69])
        W2b = W2b.at[c, LAT * j:LAT * (j + 1)].set(W2)
    b1p = jnp.tile(lp["e1"]["b"], 4).reshape(1, 4 * HID)
    b2p = jnp.tile(lp["e2"]["b"], 4).reshape(1, 4 * LAT)

    def body(hd_ref, hs_ref, aux_ref, wb_r, b1_r, w2_r, b2_r, o_ref):
        X = jnp.concatenate(
            [hd_ref[...], hs_ref[...], aux_ref[...]], axis=1)
        u = _silu(_dot(X, wb_r[...]) + b1_r[...])
        o_ref[...] = _silu(_dot(u, w2_r[...]) + b2_r[...])

    blk = pl.BlockSpec((BE4, 128), lambda i: (i, 0))
    return pl.pallas_call(
        body,
        grid=(E4 // BE4,),
        in_specs=[blk, blk, blk,
                  _full((384, 4 * HID)), _full((1, 4 * HID)),
                  _full((4 * HID, 4 * LAT)), _full((1, 4 * LAT))],
        out_specs=blk,
        out_shape=jax.ShapeDtypeStruct((EPAD4, 128), jnp.float32),
    )(hd_p, hs_p, aux_p, Wb, b1p, W2b, b2p)


def _node_update(hh, part, lp):
    Wh1 = lp["h1"]["W"]
    bh1 = lp["h1"]["b"].reshape(1, HID)
    Wh2 = lp["h2"]["W"]
    bh2 = lp["h2"]["b"].reshape(1, LAT)
    p0 = part[0]
    p1 = part[1]

    def body(hh_ref, p0_ref, p1_ref, w1_r, b1_r, w2_r, b2_r, o_ref):
        agg = p0_ref[...] + p1_ref[...]
        cat = jnp.concatenate([hh_ref[...], agg], axis=1)
        u = _silu(_dot(cat, w1_r[...]) + b1_r[...])
        o_ref[...] = hh_ref[...] + _dot(u, w2_r[...]) + b2_r[...]

    return pl.pallas_call(
        body,
        grid=(N // BN,),
        in_specs=[pl.BlockSpec((BN, LAT), lambda i: (i, 0)),
                  pl.BlockSpec((BN, LAT), lambda i: (i, 0)),
                  pl.BlockSpec((BN, LAT), lambda i: (i, 0)),
                  _full((2 * LAT, HID)), _full((1, HID)),
                  _full((HID, LAT)), _full((1, LAT))],
        out_specs=pl.BlockSpec((BN, LAT), lambda i: (i, 0)),
        out_shape=jax.ShapeDtypeStruct((N, LAT), jnp.float32),
    )(hh, p0, p1, Wh1, bh1, Wh2, bh2)


def _softplus(x):
    return jnp.maximum(x, 0.0) + jnp.log1p(jnp.exp(-jnp.abs(x)))


def _vae_prior(zgp, p):
    L1, b1 = p["l1"]["W"], p["l1"]["b"].reshape(1, MLP_HID)
    L2, b2 = p["l2"]["W"], p["l2"]["b"].reshape(1, 2 * LAT)

    def body(z_ref, l1_r, b1_r, l2_r, b2_r, loc_ref, scale_ref):
        hdn = _silu(_dot(z_ref[...], l1_r[...]) + b1_r[...])
        o = _dot(hdn, l2_r[...]) + b2_r[...]
        loc_ref[...] = o[:, 0:LAT]
        scale_ref[...] = _softplus(o[:, LAT:2 * LAT]) + 1e-4

    return pl.pallas_call(
        body,
        grid=(N // BN,),
        in_specs=[pl.BlockSpec((BN, LAT), lambda i: (i, 0)),
                  _full((LAT, MLP_HID)), _full((1, MLP_HID)),
                  _full((MLP_HID, 2 * LAT)), _full((1, 2 * LAT))],
        out_specs=[pl.BlockSpec((BN, LAT), lambda i: (i, 0)),
                   pl.BlockSpec((BN, LAT), lambda i: (i, 0))],
        out_shape=[jax.ShapeDtypeStruct((N, LAT), jnp.float32),
                   jax.ShapeDtypeStruct((N, LAT), jnp.float32)],
    )(zgp, L1, b1, L2, b2)


def _vae_inf(zg, zgp, eps, p):
    L1 = p["l1"]["W"]
    b1 = p["l1"]["b"].reshape(1, MLP_HID)
    L2, b2 = p["l2"]["W"], p["l2"]["b"].reshape(1, 2 * LAT)

    def body(zg_ref, zgp_ref, eps_ref, l1_r, b1_r, l2_r, b2_r,
             loc_ref, scale_ref, z_ref):
        cat = jnp.concatenate([zg_ref[...], zgp_ref[...]], axis=1)
        hdn = _silu(_dot(cat, l1_r[...]) + b1_r[...])
        o = _dot(hdn, l2_r[...]) + b2_r[...]
        loc = o[:, 0:LAT]
        scale = _softplus(o[:, LAT:2 * LAT]) + 1e-4
        loc_ref[...] = loc
        scale_ref[...] = scale
        z_ref[...] = loc + scale * eps_ref[...]

    return pl.pallas_call(
        body,
        grid=(N // BN,),
        in_specs=[pl.BlockSpec((BN, LAT), lambda i: (i, 0)),
                  pl.BlockSpec((BN, LAT), lambda i: (i, 0)),
                  pl.BlockSpec((BN, LAT), lambda i: (i, 0)),
                  _full((2 * LAT, MLP_HID)), _full((1, MLP_HID)),
                  _full((MLP_HID, 2 * LAT)), _full((1, 2 * LAT))],
        out_specs=[pl.BlockSpec((BN, LAT), lambda i: (i, 0)),
                   pl.BlockSpec((BN, LAT), lambda i: (i, 0)),
                   pl.BlockSpec((BN, LAT), lambda i: (i, 0))],
        out_shape=[jax.ShapeDtypeStruct((N, LAT), jnp.float32),
                   jax.ShapeDtypeStruct((N, LAT), jnp.float32),
                   jax.ShapeDtypeStruct((N, LAT), jnp.float32)],
    )(zg, zgp, eps, L1, b1, L2, b2)


def _gnn_pass(params, tables, h, ea_p, src2d, dst2d, zeros_acc):
    hh = _mlp1(h, params["embed"]["W"], params["embed"]["b"])
    pairs = []
    for t in tables:
        xs, xd = _gather2_kernel()(t, src2d, dst2d)
        pairs.append((xs.reshape(EPAD4, 128), xd.reshape(EPAD4, 128)))
    aux = _d2aux(pairs, ea_p)
    for lp in params["layers"]:
        hs, hd = _gather2_kernel()(hh, src2d, dst2d)
        m_p = _edge_mlp(hd.reshape(EPAD4, 128), hs.reshape(EPAD4, 128),
                        aux, lp)
        part = _scatter_kernel()(m_p.reshape(EPAD, LAT), dst2d, zeros_acc)
        hh = _node_update(hh, part, lp)
    return _proj(hh, params["out"]["W"], params["out"]["b"])


def kernel(x, h, edge_attr, edge_attr_partial, edge_index, partial_goal_mask,
           enc_goal_params, enc_partial_params, dec_params, inf_params,
           prior_params):
    src = edge_index[0]
    dst = edge_index[1]
    src2d = _pad_idx(src, 0)
    dst2d = _pad_idx(dst, N)
    zeros_acc = jnp.zeros((NACC, LAT), jnp.float32)

    x_pad = jnp.pad(x, ((0, 0), (0, LAT - 3)))
    xp_pad = partial_goal_mask[:, None] * x_pad
    ea_p = edge_attr.reshape(E4, 16)
    eap_p = edge_attr_partial.reshape(E4, 16)

    z_goal = _gnn_pass(enc_goal_params, (x_pad,), h, ea_p,
                       src2d, dst2d, zeros_acc)
    z_goal_partial = _gnn_pass(enc_partial_params, (xp_pad,), h,
                               eap_p, src2d, dst2d, zeros_acc)

    p_loc, p_scale = _vae_prior(z_goal_partial, prior_params)
    eps = jax.random.normal(jax.random.key(42), (N, LAT), jnp.float32)
    q_loc, q_scale, z = _vae_inf(z_goal, z_goal_partial, eps, inf_params)

    mu_x_sample = _gnn_pass(dec_params, (z, z_goal_partial), h, None,
                            src2d, dst2d, zeros_acc)
    return (mu_x_sample, q_loc, q_scale, p_loc, p_scale)
